# Initial kernel scaffold; baseline (speedup 1.0000x reference)
import jax
import jax.numpy as jnp
from jax.experimental import pallas as pl


def kernel(x, nlDists, nlInds, pixels_h, pixels_w):
    raise NotImplementedError("write your pallas kernel here")


# speedup vs baseline: 414.1304x; 414.1304x over previous
import functools

import jax
import jax.numpy as jnp
from jax import lax
from jax.experimental import pallas as pl
from jax.experimental.pallas import tpu as pltpu
from jax.experimental.pallas import tpu_sc as plsc

NC, NS, L = 2, 16, 16
NW = NC * NS
T, P_, D = 16, 8192, 192
N = T * P_
K = 4096
FPW = D // NW
CH = 2048
NCHUNK = N // CH
GRID = 23
VSTRIDE = 544

_mesh = plsc.VectorSubcoreMesh(core_axis_name="c", subcore_axis_name="s")
_params = pltpu.CompilerParams(needs_layout_passes=False, use_tc_tiling_on_sc=False)


def _lanes():
    return jnp.arange(L, dtype=jnp.int32)


@functools.partial(
    pl.kernel,
    out_type=(
        jax.ShapeDtypeStruct((D * K,), jnp.float32),
        jax.ShapeDtypeStruct((K,), jnp.float32),
        jax.ShapeDtypeStruct((N,), jnp.int32),
    ),
    mesh=_mesh,
    compiler_params=_params,
    scratch_types=[
        pltpu.VMEM((FPW * K,), jnp.float32),
        pltpu.VMEM((FPW, CH), jnp.float32),
        pltpu.VMEM((3, CH), jnp.int32),
        pltpu.VMEM((CH,), jnp.int32),
        pltpu.VMEM((K,), jnp.float32),
    ],
)
def _phase_a(x_hbm, nd_hbm, s_out, c_out, keys_out, stab, xslab, ndb, keyb, cntb):
    cid = lax.axis_index("c")
    sid = lax.axis_index("s")
    wid = sid * NC + cid
    vbase = wid * FPW
    zero_v = jnp.zeros((L,), jnp.float32)
    one_v = jnp.ones((L,), jnp.float32)

    def zs(i, _):
        stab[pl.ds(i * L, L)] = zero_v
        return 0
    lax.fori_loop(0, FPW * K // L, zs, 0)

    def zcnt(i, _):
        cntb[pl.ds(i * L, L)] = zero_v
        return 0
    lax.fori_loop(0, K // L, zcnt, 0)

    def chunk_body(ck, _):
        t = ck >> 2
        pc = ck & 3
        base = ck * CH
        pltpu.sync_copy(nd_hbm.at[pl.ds(0, 3)].at[:, pl.ds(base, CH)], ndb)
        pltpu.sync_copy(
            x_hbm.at[pl.ds(t * D + vbase, FPW)].at[:, pl.ds(pc * CH, CH)], xslab)

        def key_body(g, _):
            s = pl.ds(g * L, L)
            keyb[s] = ndb[0, s] * 256 + ndb[1, s] * 16 + ndb[2, s]
            return 0
        lax.fori_loop(0, CH // L, key_body, 0)

        @pl.when(wid == lax.rem(ck, NW))
        def _():
            pltpu.sync_copy(keyb, keys_out.at[pl.ds(base, CH)])

        def acc_body(g, _):
            s = pl.ds(g * L, L)
            kv = keyb[s]
            for f in range(FPW):
                plsc.addupdate_scatter(stab, [kv + f * K], xslab[f, s])
            return 0
        lax.fori_loop(0, CH // L, acc_body, 0)

        @pl.when(wid == 0)
        def _():
            def cnt_body(g, _):
                kv = keyb[pl.ds(g * L, L)]
                plsc.addupdate_scatter(cntb, [kv], one_v)
                return 0
            lax.fori_loop(0, CH // L, cnt_body, 0)
        return 0
    lax.fori_loop(0, NCHUNK, chunk_body, 0)

    pltpu.sync_copy(stab, s_out.at[pl.ds(vbase * K, FPW * K)])

    @pl.when(wid == 0)
    def _():
        pltpu.sync_copy(cntb, c_out)


@functools.partial(
    pl.kernel,
    out_type=jax.ShapeDtypeStruct((D, K), jnp.float32),
    mesh=_mesh,
    compiler_params=_params,
    scratch_types=[
        pltpu.VMEM((D, 256), jnp.float32),
        pltpu.VMEM((D, 256), jnp.float32),
        pltpu.VMEM((256,), jnp.float32),
        pltpu.VMEM((3 * VSTRIDE,), jnp.float32),
        pltpu.VMEM((VSTRIDE,), jnp.float32),
        pltpu.VMEM((VSTRIDE,), jnp.float32),
        pltpu.VMEM((L,), jnp.float32),
    ],
)
def _phase_b(s_hbm, c_hbm, res_hbm, p_out, stslab, ptbuf, cbuf, vid, wvid, recip, rbuf):
    cid = lax.axis_index("c")
    sid = lax.axis_index("s")
    wid = sid * NC + cid
    lanes = _lanes()
    zero_v = jnp.zeros((L,), jnp.float32)

    @pl.when(wid < T)
    def _():
        ti = wid
        pltpu.sync_copy(res_hbm, rbuf)
        rv = rbuf[pl.ds(0, L)]
        pltpu.sync_copy(s_hbm.at[pl.ds(0, D)].at[:, pl.ds(ti * 256, 256)], stslab)
        pltpu.sync_copy(c_hbm.at[pl.ds(ti * 256, 256)], cbuf)

        def zv(i, _):
            vid[pl.ds(i * L, L)] = zero_v
            return 0
        lax.fori_loop(0, 3 * VSTRIDE // L, zv, 0)

        def zw(i, _):
            wvid[pl.ds(i * L, L)] = zero_v
            return 0
        lax.fori_loop(0, VSTRIDE // L, zw, 0)

        def wfold(hi, _):
            cvec = cbuf[pl.ds(hi * L, L)]
            for i in range(8):
                for j in range(8):
                    idx = (hi + i) * GRID + j + lanes
                    plsc.addupdate_scatter(wvid, [idx], cvec)
            return 0
        lax.fori_loop(0, 16, wfold, 0)

        def vfold(hi, _):
            s = pl.ds(hi * L, L)
            for col in range(D):
                c, rem = col // 64, col % 64
                i, j = rem // 8, rem % 8
                idx = c * VSTRIDE + (hi + i) * GRID + j + lanes
                plsc.addupdate_scatter(vid, [idx], stslab[col, s])
            return 0
        lax.fori_loop(0, 16, vfold, 0)

        def rec(i, _):
            w = wvid[pl.ds(i * L, L)]
            recip[pl.ds(i * L, L)] = 1.0 / jnp.maximum(w, 1e-10)
            return 0
        lax.fori_loop(0, VSTRIDE // L, rec, 0)

        for c in range(3):
            def pm(i, _, c=c):
                s = pl.ds(c * VSTRIDE + i * L, L)
                vid[s] = vid[s] * recip[pl.ds(i * L, L)]
                return 0
            lax.fori_loop(0, VSTRIDE // L, pm, 0)

        def unf(hi, _):
            for col in range(D):
                c, rem = col // 64, col % 64
                i, j = rem // 8, rem % 8
                v = vid[pl.ds(c * VSTRIDE + (hi + i) * GRID + j, L)]
                ptbuf[col, pl.ds(hi * L, L)] = v + rv
            return 0
        lax.fori_loop(0, 16, unf, 0)

        pltpu.sync_copy(ptbuf, p_out.at[pl.ds(0, D)].at[:, pl.ds(ti * 256, 256)])


@functools.partial(
    pl.kernel,
    out_type=jax.ShapeDtypeStruct((T * D, P_), jnp.float32),
    mesh=_mesh,
    compiler_params=_params,
    scratch_types=[
        pltpu.VMEM((FPW, K), jnp.float32),
        pltpu.VMEM((FPW, CH), jnp.float32),
        pltpu.VMEM((CH,), jnp.int32),
    ],
)
def _phase_c(p_hbm, keys_hbm, out_hbm, ptslab, oslab, keyb):
    cid = lax.axis_index("c")
    sid = lax.axis_index("s")
    wid = sid * NC + cid
    vbase = wid * FPW
    pltpu.sync_copy(p_hbm.at[pl.ds(vbase, FPW)], ptslab)

    def chunk_body(ck, _):
        t = ck >> 2
        pc = ck & 3
        base = ck * CH
        pltpu.sync_copy(keys_hbm.at[pl.ds(base, CH)], keyb)

        def gat_body(g, _):
            s = pl.ds(g * L, L)
            kv = keyb[s]
            for f in range(FPW):
                fv = jnp.full((L,), f, jnp.int32)
                oslab[f, s] = plsc.load_gather(ptslab, [fv, kv])
            return 0
        lax.fori_loop(0, CH // L, gat_body, 0)

        pltpu.sync_copy(
            oslab, out_hbm.at[pl.ds(t * D + vbase, FPW)].at[:, pl.ds(pc * CH, CH)])
        return 0
    lax.fori_loop(0, NCHUNK, chunk_body, 0)


def kernel(x, nlDists, nlInds, pixels_h, pixels_w):
    t, p, hf, vf = x.shape
    xt = jnp.transpose(x, (0, 2, 3, 1)).reshape(t * vf, p)
    ndt = jnp.transpose(nlInds, (2, 3, 0, 1)).reshape(30, t * p)
    residual = ((jnp.asarray(pixels_h) - 128) + (jnp.asarray(pixels_w) - 128))
    res_splat = jnp.full((L,), 1.0, jnp.float32) * residual.astype(jnp.float32)

    st, cnt, keys = _phase_a(xt, ndt)
    p_tab = _phase_b(st.reshape(D, K), cnt, res_splat)
    outt = _phase_c(p_tab, keys)
    return jnp.transpose(outt.reshape(t, 1, vf, p), (0, 3, 1, 2))


# speedup vs baseline: 514.2966x; 1.2419x over previous
import functools

import jax
import jax.numpy as jnp
from jax import lax
from jax.experimental import pallas as pl
from jax.experimental.pallas import tpu as pltpu
from jax.experimental.pallas import tpu_sc as plsc

NC, NS, L = 2, 16, 16
NW = NC * NS
T, P_, D = 16, 8192, 192
N = T * P_
K = 4096
FPW = D // NW
CH = 2048
NCHUNK = N // CH
GRID = 23
VSTRIDE = 544

_mesh = plsc.VectorSubcoreMesh(core_axis_name="c", subcore_axis_name="s")
_params = pltpu.CompilerParams(needs_layout_passes=False, use_tc_tiling_on_sc=False)


def _lanes():
    return jnp.arange(L, dtype=jnp.int32)


@functools.partial(
    pl.kernel,
    out_type=(
        jax.ShapeDtypeStruct((D * K,), jnp.float32),
        jax.ShapeDtypeStruct((K,), jnp.float32),
        jax.ShapeDtypeStruct((N,), jnp.int32),
    ),
    mesh=_mesh,
    compiler_params=_params,
    scratch_types=[
        pltpu.VMEM((FPW * K,), jnp.float32),
        pltpu.VMEM((FPW, CH), jnp.float32),
        pltpu.VMEM((FPW, CH), jnp.float32),
        pltpu.VMEM((3, CH), jnp.int32),
        pltpu.VMEM((3, CH), jnp.int32),
        pltpu.VMEM((CH,), jnp.int32),
        pltpu.VMEM((K,), jnp.float32),
        pltpu.SemaphoreType.DMA,
        pltpu.SemaphoreType.DMA,
    ],
)
def _phase_a(x_hbm, nd_hbm, s_out, c_out, keys_out,
             stab, xs0, xs1, nd0, nd1, keyb, cntb, sem0, sem1):
    cid = lax.axis_index("c")
    sid = lax.axis_index("s")
    wid = sid * NC + cid
    vbase = wid * FPW
    zero_v = jnp.zeros((L,), jnp.float32)
    one_v = jnp.ones((L,), jnp.float32)

    def zs(i, _):
        stab[pl.ds(i * L, L)] = zero_v
        return 0
    lax.fori_loop(0, FPW * K // L, zs, 0)

    def zcnt(i, _):
        cntb[pl.ds(i * L, L)] = zero_v
        return 0
    lax.fori_loop(0, K // L, zcnt, 0)

    def issue(ck, xs, nd, sem):
        base = ck * CH
        pltpu.async_copy(nd_hbm.at[pl.ds(0, 3)].at[:, pl.ds(base, CH)], nd, sem)
        pltpu.async_copy(
            x_hbm.at[pl.ds((ck >> 2) * D + vbase, FPW)].at[:, pl.ds((ck & 3) * CH, CH)],
            xs, sem)

    def wait(xs, nd, sem):
        pltpu.make_async_copy(nd_hbm.at[pl.ds(0, 3)].at[:, pl.ds(0, CH)], nd, sem).wait()
        pltpu.make_async_copy(x_hbm.at[pl.ds(0, FPW)].at[:, pl.ds(0, CH)], xs, sem).wait()

    def compute(ck, xs, nd):
        @pl.when(wid == lax.rem(ck, NW))
        def _():
            def keyfill(g, _):
                s = pl.ds(g * L, L)
                keyb[s] = nd[0, s] * 256 + nd[1, s] * 16 + nd[2, s]
                return 0
            lax.fori_loop(0, CH // L, keyfill, 0)
            pltpu.sync_copy(keyb, keys_out.at[pl.ds(ck * CH, CH)])

        def acc_body(g, _):
            for u in range(2):
                s = pl.ds((g * 2 + u) * L, L)
                kv = nd[0, s] * 256 + nd[1, s] * 16 + nd[2, s]
                for f in range(FPW):
                    plsc.addupdate_scatter(stab, [kv + f * K], xs[f, s])
            return 0
        lax.fori_loop(0, CH // L // 2, acc_body, 0)

        @pl.when(wid == 0)
        def _():
            def cnt_body(g, _):
                s = pl.ds(g * L, L)
                kv = nd[0, s] * 256 + nd[1, s] * 16 + nd[2, s]
                plsc.addupdate_scatter(cntb, [kv], one_v)
                return 0
            lax.fori_loop(0, CH // L, cnt_body, 0)

    issue(0, xs0, nd0, sem0)

    def outer(st, _):
        ck0 = st * 2
        issue(ck0 + 1, xs1, nd1, sem1)
        wait(xs0, nd0, sem0)
        compute(ck0, xs0, nd0)

        @pl.when(st < NCHUNK // 2 - 1)
        def _():
            issue(ck0 + 2, xs0, nd0, sem0)
        wait(xs1, nd1, sem1)
        compute(ck0 + 1, xs1, nd1)
        return 0
    lax.fori_loop(0, NCHUNK // 2, outer, 0)

    pltpu.sync_copy(stab, s_out.at[pl.ds(vbase * K, FPW * K)])

    @pl.when(wid == 0)
    def _():
        pltpu.sync_copy(cntb, c_out)


@functools.partial(
    pl.kernel,
    out_type=jax.ShapeDtypeStruct((D, K), jnp.float32),
    mesh=_mesh,
    compiler_params=_params,
    scratch_types=[
        pltpu.VMEM((D, 256), jnp.float32),
        pltpu.VMEM((D, 256), jnp.float32),
        pltpu.VMEM((256,), jnp.float32),
        pltpu.VMEM((3 * VSTRIDE,), jnp.float32),
        pltpu.VMEM((VSTRIDE,), jnp.float32),
        pltpu.VMEM((VSTRIDE,), jnp.float32),
        pltpu.VMEM((L,), jnp.float32),
    ],
)
def _phase_b(s_hbm, c_hbm, res_hbm, p_out, stslab, ptbuf, cbuf, vid, wvid, recip, rbuf):
    cid = lax.axis_index("c")
    sid = lax.axis_index("s")
    wid = sid * NC + cid
    lanes = _lanes()
    zero_v = jnp.zeros((L,), jnp.float32)

    @pl.when(wid < T)
    def _():
        ti = wid
        pltpu.sync_copy(res_hbm, rbuf)
        rv = rbuf[pl.ds(0, L)]
        pltpu.sync_copy(s_hbm.at[pl.ds(0, D)].at[:, pl.ds(ti * 256, 256)], stslab)
        pltpu.sync_copy(c_hbm.at[pl.ds(ti * 256, 256)], cbuf)

        def zv(i, _):
            vid[pl.ds(i * L, L)] = zero_v
            return 0
        lax.fori_loop(0, 3 * VSTRIDE // L, zv, 0)

        def zw(i, _):
            wvid[pl.ds(i * L, L)] = zero_v
            return 0
        lax.fori_loop(0, VSTRIDE // L, zw, 0)

        def wfold(hi, _):
            cvec = cbuf[pl.ds(hi * L, L)]
            for i in range(8):
                for j in range(8):
                    idx = (hi + i) * GRID + j + lanes
                    plsc.addupdate_scatter(wvid, [idx], cvec)
            return 0
        lax.fori_loop(0, 16, wfold, 0)

        def vfold(hi, _):
            s = pl.ds(hi * L, L)
            for col in range(D):
                c, rem = col // 64, col % 64
                i, j = rem // 8, rem % 8
                idx = c * VSTRIDE + (hi + i) * GRID + j + lanes
                plsc.addupdate_scatter(vid, [idx], stslab[col, s])
            return 0
        lax.fori_loop(0, 16, vfold, 0)

        def rec(i, _):
            w = wvid[pl.ds(i * L, L)]
            recip[pl.ds(i * L, L)] = 1.0 / jnp.maximum(w, 1e-10)
            return 0
        lax.fori_loop(0, VSTRIDE // L, rec, 0)

        for c in range(3):
            def pm(i, _, c=c):
                s = pl.ds(c * VSTRIDE + i * L, L)
                vid[s] = vid[s] * recip[pl.ds(i * L, L)]
                return 0
            lax.fori_loop(0, VSTRIDE // L, pm, 0)

        def unf(hi, _):
            for col in range(D):
                c, rem = col // 64, col % 64
                i, j = rem // 8, rem % 8
                v = vid[pl.ds(c * VSTRIDE + (hi + i) * GRID + j, L)]
                ptbuf[col, pl.ds(hi * L, L)] = v + rv
            return 0
        lax.fori_loop(0, 16, unf, 0)

        pltpu.sync_copy(ptbuf, p_out.at[pl.ds(0, D)].at[:, pl.ds(ti * 256, 256)])


@functools.partial(
    pl.kernel,
    out_type=jax.ShapeDtypeStruct((T * D, P_), jnp.float32),
    mesh=_mesh,
    compiler_params=_params,
    scratch_types=[
        pltpu.VMEM((FPW * K,), jnp.float32),
        pltpu.VMEM((FPW, CH), jnp.float32),
        pltpu.VMEM((FPW, CH), jnp.float32),
        pltpu.VMEM((CH,), jnp.int32),
        pltpu.VMEM((CH,), jnp.int32),
        pltpu.SemaphoreType.DMA,
        pltpu.SemaphoreType.DMA,
        pltpu.SemaphoreType.DMA,
        pltpu.SemaphoreType.DMA,
    ],
)
def _phase_c(p_hbm, keys_hbm, out_hbm, ptslab, os0, os1, kb0, kb1,
             sk0, sk1, so0, so1):
    cid = lax.axis_index("c")
    sid = lax.axis_index("s")
    wid = sid * NC + cid
    vbase = wid * FPW
    pltpu.sync_copy(p_hbm.at[pl.ds(vbase * K, FPW * K)], ptslab)

    def issue_keys(ck, kb, sem):
        pltpu.async_copy(keys_hbm.at[pl.ds(ck * CH, CH)], kb, sem)

    def wait_keys(kb, sem):
        pltpu.make_async_copy(keys_hbm.at[pl.ds(0, CH)], kb, sem).wait()

    def out_slice(ck):
        return out_hbm.at[pl.ds((ck >> 2) * D + vbase, FPW)].at[:, pl.ds((ck & 3) * CH, CH)]

    def wait_out(os, sem):
        pltpu.make_async_copy(os, out_slice(0), sem).wait()

    def compute(kb, os):
        def gat_body(g, _):
            for u in range(2):
                s = pl.ds((g * 2 + u) * L, L)
                kv = kb[s]
                for f in range(FPW):
                    os[f, s] = plsc.load_gather(ptslab, [kv + f * K])
            return 0
        lax.fori_loop(0, CH // L // 2, gat_body, 0)

    issue_keys(0, kb0, sk0)

    def outer(st, _):
        ck0 = st * 2
        issue_keys(ck0 + 1, kb1, sk1)
        wait_keys(kb0, sk0)

        @pl.when(st > 0)
        def _():
            wait_out(os0, so0)
        compute(kb0, os0)
        pltpu.async_copy(os0, out_slice(ck0), so0)

        @pl.when(st < NCHUNK // 2 - 1)
        def _():
            issue_keys(ck0 + 2, kb0, sk0)
        wait_keys(kb1, sk1)

        @pl.when(st > 0)
        def _():
            wait_out(os1, so1)
        compute(kb1, os1)
        pltpu.async_copy(os1, out_slice(ck0 + 1), so1)
        return 0
    lax.fori_loop(0, NCHUNK // 2, outer, 0)

    wait_out(os0, so0)
    wait_out(os1, so1)


def kernel(x, nlDists, nlInds, pixels_h, pixels_w):
    t, p, hf, vf = x.shape
    xt = jnp.transpose(x, (0, 2, 3, 1)).reshape(t * vf, p)
    ndt = jnp.transpose(nlInds, (2, 3, 0, 1)).reshape(30, t * p)
    residual = ((jnp.asarray(pixels_h) - 128) + (jnp.asarray(pixels_w) - 128))
    res_splat = jnp.full((L,), 1.0, jnp.float32) * residual.astype(jnp.float32)

    st, cnt, keys = _phase_a(xt, ndt)
    p_tab = _phase_b(st.reshape(D, K), cnt, res_splat)
    outt = _phase_c(p_tab.reshape(D * K), keys)
    return jnp.transpose(outt.reshape(t, 1, vf, p), (0, 3, 1, 2))


# speedup vs baseline: 563.5983x; 1.0959x over previous
import functools

import jax
import jax.numpy as jnp
from jax import lax
from jax.experimental import pallas as pl
from jax.experimental.pallas import tpu as pltpu
from jax.experimental.pallas import tpu_sc as plsc

NC, NS, L = 2, 16, 16
NW = NC * NS
T, P_, D = 16, 8192, 192
N = T * P_
K = 4096
FPW = D // NW
CH = 2048
NCHUNK = N // CH
GRID = 23
VSTRIDE = 544

_mesh = plsc.VectorSubcoreMesh(core_axis_name="c", subcore_axis_name="s")
_params = pltpu.CompilerParams(needs_layout_passes=False, use_tc_tiling_on_sc=False)


def _lanes():
    return jnp.arange(L, dtype=jnp.int32)


@functools.partial(
    pl.kernel,
    out_type=(
        jax.ShapeDtypeStruct((D * K,), jnp.float32),
        jax.ShapeDtypeStruct((NW, K), jnp.float32),
        jax.ShapeDtypeStruct((N,), jnp.int32),
    ),
    mesh=_mesh,
    compiler_params=_params,
    scratch_types=[
        pltpu.VMEM((FPW * K,), jnp.float32),
        pltpu.VMEM((FPW, CH), jnp.float32),
        pltpu.VMEM((FPW, CH), jnp.float32),
        pltpu.VMEM((3, CH), jnp.int32),
        pltpu.VMEM((3, CH), jnp.int32),
        pltpu.VMEM((CH,), jnp.int32),
        pltpu.VMEM((K,), jnp.float32),
        pltpu.SemaphoreType.DMA,
        pltpu.SemaphoreType.DMA,
    ],
)
def _phase_a(x_hbm, nd_hbm, s_out, c_out, keys_out,
             stab, xs0, xs1, nd0, nd1, keyb, cntb, sem0, sem1):
    cid = lax.axis_index("c")
    sid = lax.axis_index("s")
    wid = sid * NC + cid
    vbase = wid * FPW
    zero_v = jnp.zeros((L,), jnp.float32)
    one_v = jnp.ones((L,), jnp.float32)

    def zs(i, _):
        stab[pl.ds(i * L, L)] = zero_v
        return 0
    lax.fori_loop(0, FPW * K // L, zs, 0)

    def zcnt(i, _):
        cntb[pl.ds(i * L, L)] = zero_v
        return 0
    lax.fori_loop(0, K // L, zcnt, 0)

    def issue(ck, xs, nd, sem):
        base = ck * CH
        pltpu.async_copy(nd_hbm.at[pl.ds(0, 3)].at[:, pl.ds(base, CH)], nd, sem)
        pltpu.async_copy(
            x_hbm.at[pl.ds((ck >> 2) * D + vbase, FPW)].at[:, pl.ds((ck & 3) * CH, CH)],
            xs, sem)

    def wait(xs, nd, sem):
        pltpu.make_async_copy(nd_hbm.at[pl.ds(0, 3)].at[:, pl.ds(0, CH)], nd, sem).wait()
        pltpu.make_async_copy(x_hbm.at[pl.ds(0, FPW)].at[:, pl.ds(0, CH)], xs, sem).wait()

    def compute(ck, xs, nd):
        @pl.when(wid == lax.rem(ck, NW))
        def _():
            def keyfill(g, _):
                s = pl.ds(g * L, L)
                keyb[s] = nd[0, s] * 256 + nd[1, s] * 16 + nd[2, s]
                return 0
            lax.fori_loop(0, CH // L, keyfill, 0)
            pltpu.sync_copy(keyb, keys_out.at[pl.ds(ck * CH, CH)])

            def cnt_body(g, _):
                kv = keyb[pl.ds(g * L, L)]
                plsc.addupdate_scatter(cntb, [kv], one_v)
                return 0
            lax.fori_loop(0, CH // L, cnt_body, 0)

        def acc_body(g, _):
            for u in range(4):
                s = pl.ds((g * 4 + u) * L, L)
                kv = nd[0, s] * 256 + nd[1, s] * 16 + nd[2, s]
                for f in range(FPW):
                    plsc.addupdate_scatter(stab, [kv + f * K], xs[f, s])
            return 0
        lax.fori_loop(0, CH // L // 4, acc_body, 0)

    issue(0, xs0, nd0, sem0)

    def outer(st, _):
        ck0 = st * 2
        issue(ck0 + 1, xs1, nd1, sem1)
        wait(xs0, nd0, sem0)
        compute(ck0, xs0, nd0)

        @pl.when(st < NCHUNK // 2 - 1)
        def _():
            issue(ck0 + 2, xs0, nd0, sem0)
        wait(xs1, nd1, sem1)
        compute(ck0 + 1, xs1, nd1)
        return 0
    lax.fori_loop(0, NCHUNK // 2, outer, 0)

    pltpu.sync_copy(stab, s_out.at[pl.ds(vbase * K, FPW * K)])
    pltpu.sync_copy(cntb, c_out.at[wid])


@functools.partial(
    pl.kernel,
    out_type=jax.ShapeDtypeStruct((D, K), jnp.float32),
    mesh=_mesh,
    compiler_params=_params,
    scratch_types=[
        pltpu.VMEM((D, 256), jnp.float32),
        pltpu.VMEM((D, 256), jnp.float32),
        pltpu.VMEM((256,), jnp.float32),
        pltpu.VMEM((NW, 256), jnp.float32),
        pltpu.VMEM((3 * VSTRIDE,), jnp.float32),
        pltpu.VMEM((VSTRIDE,), jnp.float32),
        pltpu.VMEM((VSTRIDE,), jnp.float32),
        pltpu.VMEM((L,), jnp.float32),
    ],
)
def _phase_b(s_hbm, c_hbm, res_hbm, p_out, stslab, ptbuf, cbuf, cstage, vid, wvid, recip, rbuf):
    cid = lax.axis_index("c")
    sid = lax.axis_index("s")
    wid = sid * NC + cid
    lanes = _lanes()
    zero_v = jnp.zeros((L,), jnp.float32)

    @pl.when(wid < T)
    def _():
        ti = wid
        pltpu.sync_copy(res_hbm, rbuf)
        rv = rbuf[pl.ds(0, L)]
        pltpu.sync_copy(s_hbm.at[pl.ds(0, D)].at[:, pl.ds(ti * 256, 256)], stslab)
        pltpu.sync_copy(c_hbm.at[pl.ds(0, NW)].at[:, pl.ds(ti * 256, 256)], cstage)

        def csum(g, _):
            s = pl.ds(g * L, L)
            v = cstage[0, s]
            for rr in range(1, NW):
                v = v + cstage[rr, s]
            cbuf[s] = v
            return 0
        lax.fori_loop(0, 256 // L, csum, 0)

        def zv(i, _):
            vid[pl.ds(i * L, L)] = zero_v
            return 0
        lax.fori_loop(0, 3 * VSTRIDE // L, zv, 0)

        def zw(i, _):
            wvid[pl.ds(i * L, L)] = zero_v
            return 0
        lax.fori_loop(0, VSTRIDE // L, zw, 0)

        def wfold(hi, _):
            cvec = cbuf[pl.ds(hi * L, L)]
            for i in range(8):
                for j in range(8):
                    idx = (hi + i) * GRID + j + lanes
                    plsc.addupdate_scatter(wvid, [idx], cvec)
            return 0
        lax.fori_loop(0, 16, wfold, 0)

        def vfold(hi, _):
            s = pl.ds(hi * L, L)
            for col in range(D):
                c, rem = col // 64, col % 64
                i, j = rem // 8, rem % 8
                idx = c * VSTRIDE + (hi + i) * GRID + j + lanes
                plsc.addupdate_scatter(vid, [idx], stslab[col, s])
            return 0
        lax.fori_loop(0, 16, vfold, 0)

        def rec(i, _):
            w = wvid[pl.ds(i * L, L)]
            recip[pl.ds(i * L, L)] = 1.0 / jnp.maximum(w, 1e-10)
            return 0
        lax.fori_loop(0, VSTRIDE // L, rec, 0)

        for c in range(3):
            def pm(i, _, c=c):
                s = pl.ds(c * VSTRIDE + i * L, L)
                vid[s] = vid[s] * recip[pl.ds(i * L, L)]
                return 0
            lax.fori_loop(0, VSTRIDE // L, pm, 0)

        def unf(hi, _):
            for col in range(D):
                c, rem = col // 64, col % 64
                i, j = rem // 8, rem % 8
                v = vid[pl.ds(c * VSTRIDE + (hi + i) * GRID + j, L)]
                ptbuf[col, pl.ds(hi * L, L)] = v + rv
            return 0
        lax.fori_loop(0, 16, unf, 0)

        pltpu.sync_copy(ptbuf, p_out.at[pl.ds(0, D)].at[:, pl.ds(ti * 256, 256)])


@functools.partial(
    pl.kernel,
    out_type=jax.ShapeDtypeStruct((T * D, P_), jnp.float32),
    mesh=_mesh,
    compiler_params=_params,
    scratch_types=[
        pltpu.VMEM((FPW * K,), jnp.float32),
        pltpu.VMEM((FPW, CH), jnp.float32),
        pltpu.VMEM((FPW, CH), jnp.float32),
        pltpu.VMEM((CH,), jnp.int32),
        pltpu.VMEM((CH,), jnp.int32),
        pltpu.SemaphoreType.DMA,
        pltpu.SemaphoreType.DMA,
        pltpu.SemaphoreType.DMA,
        pltpu.SemaphoreType.DMA,
    ],
)
def _phase_c(p_hbm, keys_hbm, out_hbm, ptslab, os0, os1, kb0, kb1,
             sk0, sk1, so0, so1):
    cid = lax.axis_index("c")
    sid = lax.axis_index("s")
    wid = sid * NC + cid
    vbase = wid * FPW
    pltpu.sync_copy(p_hbm.at[pl.ds(vbase * K, FPW * K)], ptslab)

    def issue_keys(ck, kb, sem):
        pltpu.async_copy(keys_hbm.at[pl.ds(ck * CH, CH)], kb, sem)

    def wait_keys(kb, sem):
        pltpu.make_async_copy(keys_hbm.at[pl.ds(0, CH)], kb, sem).wait()

    def out_slice(ck):
        return out_hbm.at[pl.ds((ck >> 2) * D + vbase, FPW)].at[:, pl.ds((ck & 3) * CH, CH)]

    def wait_out(os, sem):
        pltpu.make_async_copy(os, out_slice(0), sem).wait()

    def compute(kb, os):
        def gat_body(g, _):
            for u in range(4):
                s = pl.ds((g * 4 + u) * L, L)
                kv = kb[s]
                for f in range(FPW):
                    os[f, s] = plsc.load_gather(ptslab, [kv + f * K])
            return 0
        lax.fori_loop(0, CH // L // 4, gat_body, 0)

    issue_keys(0, kb0, sk0)

    def outer(st, _):
        ck0 = st * 2
        issue_keys(ck0 + 1, kb1, sk1)
        wait_keys(kb0, sk0)

        @pl.when(st > 0)
        def _():
            wait_out(os0, so0)
        compute(kb0, os0)
        pltpu.async_copy(os0, out_slice(ck0), so0)

        @pl.when(st < NCHUNK // 2 - 1)
        def _():
            issue_keys(ck0 + 2, kb0, sk0)
        wait_keys(kb1, sk1)

        @pl.when(st > 0)
        def _():
            wait_out(os1, so1)
        compute(kb1, os1)
        pltpu.async_copy(os1, out_slice(ck0 + 1), so1)
        return 0
    lax.fori_loop(0, NCHUNK // 2, outer, 0)

    wait_out(os0, so0)
    wait_out(os1, so1)


def kernel(x, nlDists, nlInds, pixels_h, pixels_w):
    t, p, hf, vf = x.shape
    xt = jnp.transpose(x, (0, 2, 3, 1)).reshape(t * vf, p)
    ndt = jnp.transpose(nlInds, (2, 3, 0, 1)).reshape(30, t * p)
    residual = ((jnp.asarray(pixels_h) - 128) + (jnp.asarray(pixels_w) - 128))
    res_splat = jnp.full((L,), 1.0, jnp.float32) * residual.astype(jnp.float32)

    st, cnt, keys = _phase_a(xt, ndt)
    p_tab = _phase_b(st.reshape(D, K), cnt, res_splat)
    outt = _phase_c(p_tab.reshape(D * K), keys)
    return jnp.transpose(outt.reshape(t, 1, vf, p), (0, 3, 1, 2))
